# baseline (speedup 1.0000x reference)
import numpy as np

import jax
import jax.numpy as jnp
from jax import lax
from jax.experimental import pallas as pl
from jax.experimental.pallas import tpu as pltpu
from jax.experimental.pallas import tpu_sc as plsc

N = 10000
E = 320000
FEAT = 128
HID = 128
NBR = 4
HH = 64

NCORE = 2
NSUB = 16
LANES = 16

K = 128
NBLK = 160
EPT = K * NBLK
EPT_REAL = E // NSUB
NPAD = 10240
RPT = NPAD // NSUB
ZROWS = 160

_PERM = np.empty((HID,), dtype=np.int32)
for _c in range(HID):
    _p, _l = _c // HH, _c % HH
    if _l < 32:
        _q = 32 * (_l // 16) + _l % 16
    else:
        _t = _l - 32
        _q = 32 * (_t // 16) + 16 + _t % 16
    _PERM[_c] = _p * HH + _q


MBLK = 2000


def _pack_half(rh):
    bits = lax.bitcast_convert_type(rh, jnp.int32) + jnp.int32(0x8000)
    a = lax.shift_right_logical(bits[:, :32], 16)
    bb = bits[:, 32:] & jnp.int32(-65536)
    return a | bb


def _mm_body(x_ref, w_ref, b_ref, o_ref):
    xb = x_ref[...]
    for i in range(NBR):
        r = (
            jnp.dot(xb, w_ref[i], preferred_element_type=jnp.float32)
            + b_ref[i, 0]
        )
        o_ref[0, i] = _pack_half(r[:, :HH])
        o_ref[1, i] = _pack_half(r[:, HH:])


def _linear_all(x, W, b):
    return pl.pallas_call(
        _mm_body,
        grid=(N // MBLK,),
        in_specs=[
            pl.BlockSpec((MBLK, FEAT), lambda j: (j, 0)),
            pl.BlockSpec((NBR, FEAT, HID), lambda j: (0, 0, 0)),
            pl.BlockSpec((NBR, 1, HID), lambda j: (0, 0, 0)),
        ],
        out_specs=pl.BlockSpec((2, NBR, MBLK, HH // 2),
                               lambda j: (0, 0, j, 0)),
        out_shape=jax.ShapeDtypeStruct((2, NBR, N, HH // 2), jnp.int32),
    )(x, W, b.reshape(NBR, 1, HID))


NBUF = 4
HBLK = 80


def _spmm_body(h_ref, row_ref, col_ref, val_ref, out_ref,
               acc, zbuf, rowbuf, colbuf, valbuf,
               gbuf0, gbuf1, gbuf2, gbuf3,
               sbuf0, sbuf1, sbuf2, sbuf3,
               gsem0, gsem1, gsem2, gsem3,
               ssem0, ssem1, ssem2, ssem3, isem):
    c = lax.axis_index("c")
    s = lax.axis_index("s")
    gbufs = (gbuf0, gbuf1, gbuf2, gbuf3)
    sbufs = (sbuf0, sbuf1, sbuf2, sbuf3)
    gsems = (gsem0, gsem1, gsem2, gsem3)
    ssems = (ssem0, ssem1, ssem2, ssem3)

    @plsc.parallel_loop(0, ZROWS)
    def _(r):
        for d in range(HH // LANES):
            zbuf[r, pl.ds(d * LANES, LANES)] = jnp.zeros((LANES,), jnp.float32)

    def scale_block(gb, sb, b):
        @plsc.parallel_loop(0, K // LANES)
        def _(g):
            v16 = valbuf[b, pl.ds(g * LANES, LANES)]
            for j in range(LANES):
                e = g * LANES + j
                bc = jnp.zeros((LANES,), jnp.float32) + v16[j]
                for m in range(HH // 32):
                    vi = gb[e, pl.ds(m * LANES, LANES)]
                    lo = lax.bitcast_convert_type(vi << 16, jnp.float32)
                    hi = lax.bitcast_convert_type(
                        vi & jnp.int32(-65536), jnp.float32)
                    sb[e, pl.ds(m * 32, LANES)] = lo * bc
                    sb[e, pl.ds(m * 32 + LANES, LANES)] = hi * bc

    def round_body(bp, _):
        bi = lax.shift_right_logical(bp, 1)
        p = jnp.bitwise_and(bp, 1)
        i_br = c + NCORE * bi
        chunk = pl.multiple_of((i_br * NSUB + s) * NBLK, NBLK)
        if True:
            tbl = h_ref.at[p]

            for z in range(RPT // ZROWS):
                pltpu.async_copy(
                    zbuf, acc.at[pl.ds(s * RPT + z * ZROWS, ZROWS)], isem)
            for z in range(RPT // ZROWS):
                pltpu.make_async_copy(
                    zbuf, acc.at[pl.ds(s * RPT + z * ZROWS, ZROWS)],
                    isem).wait()
            plsc.subcore_barrier()

            def start_g(b_idx, gb, sem):
                pltpu.async_copy(tbl.at[colbuf.at[b_idx]], gb, sem)

            def wait_g(gb, sem):
                pltpu.make_async_copy(tbl.at[colbuf.at[0]], gb, sem).wait()

            def start_s(b_idx, sb, sem):
                pltpu.async_copy(sb, acc.at[rowbuf.at[b_idx]], sem, add=True)

            def wait_s(sb, sem):
                pltpu.make_async_copy(sb, acc.at[rowbuf.at[0]], sem).wait()

            def piece_body(hb, _):
                off = pl.multiple_of(chunk + hb * HBLK, HBLK)
                pltpu.async_copy(row_ref.at[pl.ds(off, HBLK)], rowbuf, isem)
                pltpu.async_copy(col_ref.at[pl.ds(off, HBLK)], colbuf, isem)
                pltpu.async_copy(val_ref.at[pl.ds(off, HBLK)], valbuf, isem)
                pltpu.make_async_copy(
                    row_ref.at[pl.ds(off, HBLK)], rowbuf, isem).wait()
                pltpu.make_async_copy(
                    col_ref.at[pl.ds(off, HBLK)], colbuf, isem).wait()
                pltpu.make_async_copy(
                    val_ref.at[pl.ds(off, HBLK)], valbuf, isem).wait()

                start_g(0, gbuf0, gsem0)
                start_g(1, gbuf1, gsem1)
                start_g(2, gbuf2, gsem2)

                def quad_body(pi, _):
                    for q in range(NBUF):
                        b = NBUF * pi + q
                        q3 = (q + 3) % NBUF
                        bn = jnp.where(b + 3 < HBLK, b + 3, 0)
                        start_g(bn, gbufs[q3], gsems[q3])
                        wait_g(gbufs[q], gsems[q])
                        @pl.when(b >= NBUF)
                        def _():
                            wait_s(sbufs[q], ssems[q])
                        scale_block(gbufs[q], sbufs[q], b)
                        start_s(b, sbufs[q], ssems[q])
                    return 0

                lax.fori_loop(0, HBLK // NBUF, quad_body, 0)
                wait_g(gbuf0, gsem0)
                wait_g(gbuf1, gsem1)
                wait_g(gbuf2, gsem2)
                for q in range(NBUF):
                    wait_s(sbufs[q], ssems[q])
                return 0

            lax.fori_loop(0, NBLK // HBLK, piece_body, 0)
            plsc.subcore_barrier()

            pltpu.sync_copy(
                acc.at[pl.ds(s * RPT, RPT)],
                out_ref.at[p, i_br, pl.ds(s * RPT, RPT)],
            )
        return 0

    lax.fori_loop(0, 2 * (NBR // NCORE), round_body, 0)


def _spmm_all(h_packed, rows2, cols2, vals2):
    mesh = plsc.VectorSubcoreMesh(core_axis_name="c", subcore_axis_name="s")
    fn = pl.kernel(
        _spmm_body,
        out_type=jax.ShapeDtypeStruct((2, NBR, NPAD, HH), jnp.float32),
        mesh=mesh,
        scratch_types=[
            pltpu.VMEM_SHARED((NPAD, HH), jnp.float32),
            pltpu.VMEM((ZROWS, HH), jnp.float32),
            pltpu.VMEM((HBLK, K), jnp.int32),
            pltpu.VMEM((HBLK, K), jnp.int32),
            pltpu.VMEM((HBLK, K), jnp.float32),
            pltpu.VMEM((K, HH // 2), jnp.int32),
            pltpu.VMEM((K, HH // 2), jnp.int32),
            pltpu.VMEM((K, HH // 2), jnp.int32),
            pltpu.VMEM((K, HH // 2), jnp.int32),
            pltpu.VMEM((K, HH), jnp.float32),
            pltpu.VMEM((K, HH), jnp.float32),
            pltpu.VMEM((K, HH), jnp.float32),
            pltpu.VMEM((K, HH), jnp.float32),
            pltpu.SemaphoreType.DMA,
            pltpu.SemaphoreType.DMA,
            pltpu.SemaphoreType.DMA,
            pltpu.SemaphoreType.DMA,
            pltpu.SemaphoreType.DMA,
            pltpu.SemaphoreType.DMA,
            pltpu.SemaphoreType.DMA,
            pltpu.SemaphoreType.DMA,
            pltpu.SemaphoreType.DMA,
        ],
        compiler_params=pltpu.CompilerParams(use_tc_tiling_on_sc=False),
    )
    return fn(h_packed, rows2, cols2, vals2)


EBLK = 2000


def _elu_body(a_ref, o_ref):
    parts = []
    for i in range(NBR):
        for p2 in range(2):
            parts.append(a_ref[p2, i])
    v = jnp.concatenate(parts, axis=-1)
    o_ref[...] = jnp.where(v > 0.0, v, jnp.exp(v) - 1.0)


def _elu_concat(agg_halves):
    return pl.pallas_call(
        _elu_body,
        grid=(N // EBLK,),
        in_specs=[pl.BlockSpec((2, NBR, EBLK, HH), lambda j: (0, 0, j, 0))],
        out_specs=pl.BlockSpec((EBLK, NBR * HID), lambda j: (j, 0)),
        out_shape=jax.ShapeDtypeStruct((N, NBR * HID), jnp.float32),
    )(agg_halves)


def _chunk_edges(a):
    a3 = a.reshape(NBR, NSUB, EPT_REAL)
    a3 = jnp.pad(a3, ((0, 0), (0, 0), (0, EPT - EPT_REAL)))
    return a3.reshape(NBR * NSUB * NBLK, K)


@jax.jit
def kernel(x, adjs_edge_index, adjs_values, W, b):
    rows = adjs_edge_index[:, 0, :].astype(jnp.int32)
    cols = adjs_edge_index[:, 1, :].astype(jnp.int32)
    cols = cols + (jnp.arange(NBR, dtype=jnp.int32) * N)[:, None]
    rows2 = _chunk_edges(rows)
    cols2 = _chunk_edges(cols)
    vals2 = _chunk_edges(adjs_values.astype(jnp.float32))

    perm = jnp.asarray(_PERM)
    h = _linear_all(x, W[:, :, perm], b[:, perm])
    agg = _spmm_all(h.reshape(2, NBR * N, HH // 2), rows2, cols2, vals2)
    return _elu_concat(agg)


# speedup vs baseline: 1.0016x; 1.0016x over previous
import numpy as np

import jax
import jax.numpy as jnp
from jax import lax
from jax.experimental import pallas as pl
from jax.experimental.pallas import tpu as pltpu
from jax.experimental.pallas import tpu_sc as plsc

N = 10000
E = 320000
FEAT = 128
HID = 128
NBR = 4
HH = 64

NCORE = 2
NSUB = 16
LANES = 16

K = 128
NBLK = 160
EPT = K * NBLK
EPT_REAL = E // NSUB
NPAD = 10240
RPT = NPAD // NSUB
ZROWS = 160

_PERM = np.empty((HID,), dtype=np.int32)
for _c in range(HID):
    _p, _l = _c // HH, _c % HH
    if _l < 32:
        _q = 32 * (_l // 16) + _l % 16
    else:
        _t = _l - 32
        _q = 32 * (_t // 16) + 16 + _t % 16
    _PERM[_c] = _p * HH + _q


MBLK = 1000


def _pack_half(rh):
    bits = lax.bitcast_convert_type(rh, jnp.int32) + jnp.int32(0x8000)
    a = lax.shift_right_logical(bits[:, :32], 16)
    bb = bits[:, 32:] & jnp.int32(-65536)
    return a | bb


def _mm_body(x_ref, w_ref, b_ref, o_ref):
    xb = x_ref[...]
    for i in range(NBR):
        r = (
            jnp.dot(xb, w_ref[i], preferred_element_type=jnp.float32)
            + b_ref[i, 0]
        )
        o_ref[0, i] = _pack_half(r[:, :HH])
        o_ref[1, i] = _pack_half(r[:, HH:])


def _linear_all(x, W, b):
    return pl.pallas_call(
        _mm_body,
        grid=(N // MBLK,),
        in_specs=[
            pl.BlockSpec((MBLK, FEAT), lambda j: (j, 0)),
            pl.BlockSpec((NBR, FEAT, HID), lambda j: (0, 0, 0)),
            pl.BlockSpec((NBR, 1, HID), lambda j: (0, 0, 0)),
        ],
        out_specs=pl.BlockSpec((2, NBR, MBLK, HH // 2),
                               lambda j: (0, 0, j, 0)),
        out_shape=jax.ShapeDtypeStruct((2, NBR, N, HH // 2), jnp.int32),
    )(x, W, b.reshape(NBR, 1, HID))


NBUF = 4
HBLK = 80


def _spmm_body(h_ref, row_ref, col_ref, val_ref, out_ref,
               acc, zbuf, rowbuf, colbuf, valbuf,
               gbuf0, gbuf1, gbuf2, gbuf3,
               sbuf0, sbuf1, sbuf2, sbuf3,
               gsem0, gsem1, gsem2, gsem3,
               ssem0, ssem1, ssem2, ssem3, isem):
    c = lax.axis_index("c")
    s = lax.axis_index("s")
    gbufs = (gbuf0, gbuf1, gbuf2, gbuf3)
    sbufs = (sbuf0, sbuf1, sbuf2, sbuf3)
    gsems = (gsem0, gsem1, gsem2, gsem3)
    ssems = (ssem0, ssem1, ssem2, ssem3)

    @plsc.parallel_loop(0, ZROWS)
    def _(r):
        for d in range(HH // LANES):
            zbuf[r, pl.ds(d * LANES, LANES)] = jnp.zeros((LANES,), jnp.float32)

    def scale_block(gb, sb, b):
        @plsc.parallel_loop(0, K // LANES)
        def _(g):
            v16 = valbuf[b, pl.ds(g * LANES, LANES)]
            for j in range(LANES):
                e = g * LANES + j
                bc = jnp.zeros((LANES,), jnp.float32) + v16[j]
                for m in range(HH // 32):
                    vi = gb[e, pl.ds(m * LANES, LANES)]
                    lo = lax.bitcast_convert_type(vi << 16, jnp.float32)
                    hi = lax.bitcast_convert_type(
                        vi & jnp.int32(-65536), jnp.float32)
                    sb[e, pl.ds(m * 32, LANES)] = lo * bc
                    sb[e, pl.ds(m * 32 + LANES, LANES)] = hi * bc

    def round_body(bp, _):
        bi = lax.shift_right_logical(bp, 1)
        p = jnp.bitwise_and(bp, 1)
        i_br = c + NCORE * bi
        chunk = pl.multiple_of((i_br * NSUB + s) * NBLK, NBLK)
        if True:
            tbl = h_ref.at[p]

            for z in range(RPT // ZROWS):
                pltpu.async_copy(
                    zbuf, acc.at[pl.ds(s * RPT + z * ZROWS, ZROWS)], isem)
            for z in range(RPT // ZROWS):
                pltpu.make_async_copy(
                    zbuf, acc.at[pl.ds(s * RPT + z * ZROWS, ZROWS)],
                    isem).wait()
            plsc.subcore_barrier()

            def start_g(b_idx, gb, sem):
                pltpu.async_copy(tbl.at[colbuf.at[b_idx]], gb, sem)

            def wait_g(gb, sem):
                pltpu.make_async_copy(tbl.at[colbuf.at[0]], gb, sem).wait()

            def start_s(b_idx, sb, sem):
                pltpu.async_copy(sb, acc.at[rowbuf.at[b_idx]], sem, add=True)

            def wait_s(sb, sem):
                pltpu.make_async_copy(sb, acc.at[rowbuf.at[0]], sem).wait()

            def piece_body(hb, _):
                off = pl.multiple_of(chunk + hb * HBLK, HBLK)
                pltpu.async_copy(row_ref.at[pl.ds(off, HBLK)], rowbuf, isem)
                pltpu.async_copy(col_ref.at[pl.ds(off, HBLK)], colbuf, isem)
                pltpu.async_copy(val_ref.at[pl.ds(off, HBLK)], valbuf, isem)
                pltpu.make_async_copy(
                    row_ref.at[pl.ds(off, HBLK)], rowbuf, isem).wait()
                pltpu.make_async_copy(
                    col_ref.at[pl.ds(off, HBLK)], colbuf, isem).wait()
                pltpu.make_async_copy(
                    val_ref.at[pl.ds(off, HBLK)], valbuf, isem).wait()

                start_g(0, gbuf0, gsem0)
                start_g(1, gbuf1, gsem1)
                start_g(2, gbuf2, gsem2)

                def quad_body(pi, _):
                    for q in range(NBUF):
                        b = NBUF * pi + q
                        q3 = (q + 3) % NBUF
                        bn = jnp.where(b + 3 < HBLK, b + 3, 0)
                        start_g(bn, gbufs[q3], gsems[q3])
                        wait_g(gbufs[q], gsems[q])
                        @pl.when(b >= NBUF)
                        def _():
                            wait_s(sbufs[q], ssems[q])
                        scale_block(gbufs[q], sbufs[q], b)
                        start_s(b, sbufs[q], ssems[q])
                    return 0

                lax.fori_loop(0, HBLK // NBUF, quad_body, 0)
                wait_g(gbuf0, gsem0)
                wait_g(gbuf1, gsem1)
                wait_g(gbuf2, gsem2)
                for q in range(NBUF):
                    wait_s(sbufs[q], ssems[q])
                return 0

            lax.fori_loop(0, NBLK // HBLK, piece_body, 0)
            plsc.subcore_barrier()

            pltpu.sync_copy(
                acc.at[pl.ds(s * RPT, RPT)],
                out_ref.at[p, i_br, pl.ds(s * RPT, RPT)],
            )
            plsc.subcore_barrier()
        return 0

    lax.fori_loop(0, 2 * (NBR // NCORE), round_body, 0)


def _spmm_all(h_packed, rows2, cols2, vals2):
    mesh = plsc.VectorSubcoreMesh(core_axis_name="c", subcore_axis_name="s")
    fn = pl.kernel(
        _spmm_body,
        out_type=jax.ShapeDtypeStruct((2, NBR, NPAD, HH), jnp.float32),
        mesh=mesh,
        scratch_types=[
            pltpu.VMEM_SHARED((NPAD, HH), jnp.float32),
            pltpu.VMEM((ZROWS, HH), jnp.float32),
            pltpu.VMEM((HBLK, K), jnp.int32),
            pltpu.VMEM((HBLK, K), jnp.int32),
            pltpu.VMEM((HBLK, K), jnp.float32),
            pltpu.VMEM((K, HH // 2), jnp.int32),
            pltpu.VMEM((K, HH // 2), jnp.int32),
            pltpu.VMEM((K, HH // 2), jnp.int32),
            pltpu.VMEM((K, HH // 2), jnp.int32),
            pltpu.VMEM((K, HH), jnp.float32),
            pltpu.VMEM((K, HH), jnp.float32),
            pltpu.VMEM((K, HH), jnp.float32),
            pltpu.VMEM((K, HH), jnp.float32),
            pltpu.SemaphoreType.DMA,
            pltpu.SemaphoreType.DMA,
            pltpu.SemaphoreType.DMA,
            pltpu.SemaphoreType.DMA,
            pltpu.SemaphoreType.DMA,
            pltpu.SemaphoreType.DMA,
            pltpu.SemaphoreType.DMA,
            pltpu.SemaphoreType.DMA,
            pltpu.SemaphoreType.DMA,
        ],
        compiler_params=pltpu.CompilerParams(use_tc_tiling_on_sc=False),
    )
    return fn(h_packed, rows2, cols2, vals2)


EBLK = 2000


def _elu_body(a_ref, o_ref):
    parts = []
    for i in range(NBR):
        for p2 in range(2):
            parts.append(a_ref[p2, i])
    v = jnp.concatenate(parts, axis=-1)
    o_ref[...] = jnp.where(v > 0.0, v, jnp.exp(v) - 1.0)


def _elu_concat(agg_halves):
    return pl.pallas_call(
        _elu_body,
        grid=(N // EBLK,),
        in_specs=[pl.BlockSpec((2, NBR, EBLK, HH), lambda j: (0, 0, j, 0))],
        out_specs=pl.BlockSpec((EBLK, NBR * HID), lambda j: (j, 0)),
        out_shape=jax.ShapeDtypeStruct((N, NBR * HID), jnp.float32),
    )(agg_halves)


def _chunk_edges(a):
    a3 = a.reshape(NBR, NSUB, EPT_REAL)
    a3 = jnp.pad(a3, ((0, 0), (0, 0), (0, EPT - EPT_REAL)))
    return a3.reshape(NBR * NSUB * NBLK, K)


@jax.jit
def kernel(x, adjs_edge_index, adjs_values, W, b):
    rows = adjs_edge_index[:, 0, :].astype(jnp.int32)
    cols = adjs_edge_index[:, 1, :].astype(jnp.int32)
    cols = cols + (jnp.arange(NBR, dtype=jnp.int32) * N)[:, None]
    rows2 = _chunk_edges(rows)
    cols2 = _chunk_edges(cols)
    vals2 = _chunk_edges(adjs_values.astype(jnp.float32))

    perm = jnp.asarray(_PERM)
    h = _linear_all(x, W[:, :, perm], b[:, perm])
    agg = _spmm_all(h.reshape(2, NBR * N, HH // 2), rows2, cols2, vals2)
    return _elu_concat(agg)
